# baseline (speedup 1.0000x reference)
import jax
import jax.numpy as jnp
from jax.experimental import pallas as pl
from jax.experimental.pallas import tpu as pltpu

_CTR = 128


def _body(x_ref, probs_ref, idx_ref, topp_ref, aux_ref):
    s = jnp.sum(x_ref[0:8, 0:128], axis=0, keepdims=True)[:, 0:8]
    probs_ref[...] = jnp.zeros_like(probs_ref) + s
    idx_ref[...] = jnp.zeros_like(idx_ref)
    topp_ref[...] = jnp.zeros_like(topp_ref)
    aux_ref[...] = jnp.zeros_like(aux_ref)


def kernel(x, W):
    b, sq, d = x.shape
    total = b * sq
    rows = total // 8
    wide = 8 * d
    num = rows // _CTR
    xw = x.reshape(rows, wide)
    probs, idx, topp, aux = pl.pallas_call(
        _body,
        grid=(num,),
        in_specs=[pl.BlockSpec((_CTR, wide), lambda i: (i, 0))],
        out_specs=[
            pl.BlockSpec((_CTR * 8, 8), lambda i: (i, 0)),
            pl.BlockSpec((_CTR * 8, 2), lambda i: (i, 0)),
            pl.BlockSpec((_CTR * 8, 2), lambda i: (i, 0)),
            pl.BlockSpec((1, 1), lambda i: (0, 0)),
        ],
        out_shape=[
            jax.ShapeDtypeStruct((total, 8), jnp.float32),
            jax.ShapeDtypeStruct((total, 2), jnp.int32),
            jax.ShapeDtypeStruct((total, 2), jnp.float32),
            jax.ShapeDtypeStruct((1, 1), jnp.float32),
        ],
        compiler_params=pltpu.CompilerParams(
            dimension_semantics=("arbitrary",),
        ),
    )(xw)
    return (probs.reshape(b, sq, 8),
            idx.reshape(b, sq, 2),
            topp.reshape(b, sq, 2),
            aux[0, 0])


# speedup vs baseline: 3.3551x; 3.3551x over previous
import functools

import jax
import jax.numpy as jnp
from jax.experimental import pallas as pl
from jax.experimental.pallas import tpu as pltpu

_D_MODEL = 2048
_NUM_EXPERTS = 8
_TOP_K = 2
_AUX_LOSS_WEIGHT = 0.01
_DPSL_PRIOR = 0.125

_CT = 512
_NBUF = 5
_NSUB = 1


def _route_chunk(x, wt):
    logits = jnp.dot(x, wt, preferred_element_type=jnp.float32)
    m = jnp.max(logits, axis=-1, keepdims=True)
    e = jnp.exp(logits - m)
    s = jnp.sum(e, axis=-1, keepdims=True)
    probs = e / s
    iota = jax.lax.broadcasted_iota(jnp.int32, probs.shape, 1)
    p1 = jnp.max(probs, axis=-1, keepdims=True)
    i1 = jnp.min(jnp.where(probs == p1, iota, _NUM_EXPERTS),
                 axis=-1, keepdims=True)
    masked = jnp.where(iota == i1, -jnp.inf, probs)
    p2 = jnp.max(masked, axis=-1, keepdims=True)
    i2 = jnp.min(jnp.where(masked == p2, iota, _NUM_EXPERTS),
                 axis=-1, keepdims=True)
    denom = p1 + p2
    idx = jnp.concatenate([i1, i2], axis=-1)
    topp = jnp.concatenate([p1 / denom, p2 / denom], axis=-1)
    psum = jnp.sum(probs, axis=0, keepdims=True)
    cnt = jnp.sum((iota == i1).astype(jnp.float32)
                  + (iota == i2).astype(jnp.float32),
                  axis=0, keepdims=True)
    return probs, idx, topp, psum, cnt


def _router_body(num_chunks,
                 x_hbm, wt_ref,
                 probs_ref, idx_ref, topp_ref, psum_ref, cnt_ref,
                 buf, sems):
    i = pl.program_id(0)

    def copy_in(chunk, slot):
        sub = _CT // _NSUB
        for j in range(_NSUB):
            pltpu.make_async_copy(
                x_hbm.at[pl.ds(chunk * _CT + j * sub, sub), :],
                buf.at[slot, pl.ds(j * sub, sub), :],
                sems.at[slot],
            ).start()

    @pl.when(i == 0)
    def _warmup():
        for j in range(_NBUF - 1):
            copy_in(j, j)

    nxt = i + _NBUF - 1

    @pl.when(nxt < num_chunks)
    def _prefetch():
        copy_in(nxt, jax.lax.rem(nxt, _NBUF))

    slot = jax.lax.rem(i, _NBUF)
    pltpu.make_async_copy(
        x_hbm.at[pl.ds(i * _CT, _CT), :],
        buf.at[slot],
        sems.at[slot],
    ).wait()

    probs, idx, topp, psum, cnt = _route_chunk(buf[slot], wt_ref[...])
    probs_ref[...] = probs
    idx_ref[...] = idx
    topp_ref[...] = topp
    psum_ref[...] = psum[None]
    cnt_ref[...] = cnt[None]


def _aux_body(total_tokens, psum_ref, cnt_ref, aux_ref):
    inv_t = 1.0 / total_tokens
    P_i = jnp.sum(psum_ref[...], axis=0, keepdims=True) * inv_t
    f_i = jnp.sum(cnt_ref[...], axis=0, keepdims=True) * (inv_t / _TOP_K)
    lb = jnp.sum(f_i * P_i, axis=-1, keepdims=True) * _NUM_EXPERTS
    prior = _DPSL_PRIOR
    dpsl = jnp.sum(prior * (jnp.log(prior) - jnp.log(P_i)),
                   axis=-1, keepdims=True)
    aux_ref[...] = _AUX_LOSS_WEIGHT * (lb + dpsl)


def kernel(x, W):
    b, s, d = x.shape
    total = b * s
    num_chunks = total // _CT
    xf = x.reshape(total, d)
    probs, idx, topp, psum, cnt = pl.pallas_call(
        functools.partial(_router_body, num_chunks),
        grid=(num_chunks,),
        in_specs=[
            pl.BlockSpec(memory_space=pltpu.MemorySpace.HBM),
            pl.BlockSpec((d, _NUM_EXPERTS), lambda i: (0, 0)),
        ],
        out_specs=[
            pl.BlockSpec((_CT, _NUM_EXPERTS), lambda i: (i, 0)),
            pl.BlockSpec((_CT, _TOP_K), lambda i: (i, 0)),
            pl.BlockSpec((_CT, _TOP_K), lambda i: (i, 0)),
            pl.BlockSpec((1, 1, _NUM_EXPERTS), lambda i: (i, 0, 0)),
            pl.BlockSpec((1, 1, _NUM_EXPERTS), lambda i: (i, 0, 0)),
        ],
        out_shape=[
            jax.ShapeDtypeStruct((total, _NUM_EXPERTS), jnp.float32),
            jax.ShapeDtypeStruct((total, _TOP_K), jnp.int32),
            jax.ShapeDtypeStruct((total, _TOP_K), jnp.float32),
            jax.ShapeDtypeStruct((num_chunks, 1, _NUM_EXPERTS), jnp.float32),
            jax.ShapeDtypeStruct((num_chunks, 1, _NUM_EXPERTS), jnp.float32),
        ],
        scratch_shapes=[
            pltpu.VMEM((_NBUF, _CT, _D_MODEL), jnp.float32),
            pltpu.SemaphoreType.DMA((_NBUF,)),
        ],
        compiler_params=pltpu.CompilerParams(
            dimension_semantics=("arbitrary",),
        ),
    )(xf, W.T)

    aux = pl.pallas_call(
        functools.partial(_aux_body, total),
        out_shape=jax.ShapeDtypeStruct((1, 1), jnp.float32),
    )(psum.reshape(num_chunks, _NUM_EXPERTS),
      cnt.reshape(num_chunks, _NUM_EXPERTS))

    return (probs.reshape(b, s, _NUM_EXPERTS),
            idx.reshape(b, s, _TOP_K),
            topp.reshape(b, s, _TOP_K),
            aux[0, 0])
